# baseline (speedup 1.0000x reference)
import functools

import jax
import jax.numpy as jnp
from jax.experimental import pallas as pl
from jax.experimental.pallas import tpu as pltpu

N_ROWS = 10000
D = 128
TILE = 2000
NJ = N_ROWS // TILE
INV_N = 1.0 / N_ROWS


def _tile_body(xs_ref, xt_ref, w_ref, hs_ref, ht_ref, mom_ref):
    i = pl.program_id(1)

    def run(x_ref, w, h_out_ref):
        h = jnp.dot(x_ref[...], w, preferred_element_type=jnp.float32)
        h_out_ref[...] = h
        h2 = h * h
        h3 = h2 * h
        h4 = h2 * h2
        h5 = h4 * h
        part = jnp.concatenate(
            [
                jnp.sum(h, axis=0, keepdims=True),
                jnp.sum(h2, axis=0, keepdims=True),
                jnp.sum(h3, axis=0, keepdims=True),
                jnp.sum(h4, axis=0, keepdims=True),
                jnp.sum(h5, axis=0, keepdims=True),
            ],
            axis=0,
        )
        mom_ref[...] = part.reshape(1, 1, 5, D)

    @pl.when(i == 0)
    def _s():
        run(xs_ref, w_ref[0], hs_ref)

    @pl.when(i == 1)
    def _t():
        run(xt_ref, w_ref[1], ht_ref)


def _loss_body(mom_ref, loss_ref):
    a = jnp.sum(mom_ref[...], axis=0) * INV_N

    def central(rows):
        m1 = rows[0:1, :]
        m2 = rows[1:2, :]
        m3 = rows[2:3, :]
        m4 = rows[3:4, :]
        m5 = rows[4:5, :]
        c2 = m2 - m1 * m1
        c3 = m3 - 3.0 * m1 * m2 + 2.0 * m1**3
        c4 = m4 - 4.0 * m1 * m3 + 6.0 * m1**2 * m2 - 3.0 * m1**4
        c5 = (
            m5
            - 5.0 * m1 * m4
            + 10.0 * m1**2 * m3
            - 10.0 * m1**3 * m2
            + 4.0 * m1**5
        )
        return m1, c2, c3, c4, c5

    s_moms = central(a[0])
    t_moms = central(a[1])
    loss = jnp.zeros((1, 1), jnp.float32)
    for s_m, t_m in zip(s_moms, t_moms):
        d = s_m - t_m
        loss = loss + jnp.sqrt(jnp.sum(d * d, keepdims=True))
    loss_ref[...] = loss


@functools.partial(jax.jit, static_argnames=())
def _run(feat_s, feat_t, w_stacked):
    tile_fn = pl.pallas_call(
        _tile_body,
        grid=(NJ, 2),
        in_specs=[
            pl.BlockSpec((TILE, D), lambda j, i: (j, 0)),
            pl.BlockSpec((TILE, D), lambda j, i: (j, 0)),
            pl.BlockSpec((2, D, D), lambda j, i: (0, 0, 0)),
        ],
        out_specs=[
            pl.BlockSpec((TILE, D), lambda j, i: (j, 0)),
            pl.BlockSpec((TILE, D), lambda j, i: (j, 0)),
            pl.BlockSpec((1, 1, 5, D), lambda j, i: (j, i, 0, 0)),
        ],
        out_shape=[
            jax.ShapeDtypeStruct((N_ROWS, D), jnp.float32),
            jax.ShapeDtypeStruct((N_ROWS, D), jnp.float32),
            jax.ShapeDtypeStruct((NJ, 2, 5, D), jnp.float32),
        ],
        compiler_params=pltpu.CompilerParams(
            dimension_semantics=("parallel", "arbitrary"),
        ),
    )
    h_s, h_t, moms = tile_fn(feat_s, feat_t, w_stacked)

    loss_fn = pl.pallas_call(
        _loss_body,
        out_shape=jax.ShapeDtypeStruct((1, 1), jnp.float32),
    )
    loss = loss_fn(moms)
    return h_s, h_t, loss


def kernel(feat_s, feat_t, W_s, W_t, edge_index):
    del edge_index
    w_stacked = jnp.stack([W_s, W_t])
    h_s, h_t, loss = _run(feat_s, feat_t, w_stacked)
    return (h_s, h_t, loss[0, 0])
